# baseline (speedup 1.0000x reference)
import jax
import jax.numpy as jnp
from jax import lax
from jax.experimental import pallas as pl
from jax.experimental.pallas import tpu as pltpu
from jax.experimental.pallas import tpu_sc as plsc

B = 4096
S = 200
D = 64
DP = 128
NC = 2
NS = 16
NW = NC * NS
BBLK = B // NW
LANES = 16
BG = BBLK // LANES


def _embed_body(idx_hbm, table_hbm, pos_hbm, out_hbm, idx_v, rows_v, obuf_v, pos_v, sem):
    wid = lax.axis_index("s") * NC + lax.axis_index("c")
    b0 = wid * BBLK
    pltpu.sync_copy(pos_hbm, pos_v)
    lane = lax.iota(jnp.int32, LANES)

    def s_body(s, carry):
        pltpu.sync_copy(idx_hbm.at[s, pl.ds(b0, BBLK)], idx_v)
        pltpu.async_copy(table_hbm.at[idx_v], rows_v, sem).wait()

        for dg in range(D // LANES):
            pg = pos_v[s, pl.ds(dg * LANES, LANES)]
            for dl in range(LANES):
                d = dg * LANES + dl
                pv = jnp.broadcast_to(pg[dl], (LANES,))
                col = jnp.full((LANES,), d, jnp.int32)
                for j in range(BG):
                    vals = plsc.load_gather(rows_v, [j * LANES + lane, col])
                    obuf_v[d, pl.ds(j * LANES, LANES)] = vals + pv

        pltpu.sync_copy(obuf_v, out_hbm.at[s, :, pl.ds(b0, BBLK)])
        return carry

    lax.fori_loop(0, S, s_body, 0)


def kernel(batch_seqs, item_emb, pos_weight):
    idx_t = batch_seqs.T
    table_p = jnp.pad(item_emb, ((0, 0), (0, DP - D)))
    k = pl.kernel(
        _embed_body,
        out_type=jax.ShapeDtypeStruct((S, D, B), jnp.float32),
        mesh=plsc.VectorSubcoreMesh(core_axis_name="c", subcore_axis_name="s"),
        compiler_params=pltpu.CompilerParams(
            use_tc_tiling_on_sc=True, needs_layout_passes=False
        ),
        scratch_types=[
            pltpu.VMEM((BBLK,), jnp.int32),
            pltpu.VMEM((BBLK, DP), jnp.float32),
            pltpu.VMEM((D, BBLK), jnp.float32),
            pltpu.VMEM((S, D), jnp.float32),
            pltpu.SemaphoreType.DMA,
        ],
    )
    out_phys = k(idx_t, table_p, pos_weight)
    return jnp.transpose(out_phys, (2, 0, 1))


# speedup vs baseline: 1.3125x; 1.3125x over previous
import jax
import jax.numpy as jnp
from jax import lax
from jax.experimental import pallas as pl
from jax.experimental.pallas import tpu as pltpu
from jax.experimental.pallas import tpu_sc as plsc

B = 4096
S = 200
D = 64
DP = 128
NC = 2
NS = 16
NW = NC * NS
BBLK = B // NW
LANES = 16
BG = BBLK // LANES
DG = D // LANES


def _embed_body(idx_hbm, table_hbm, pos_hbm, out_hbm,
                idx_v, rows0, rows1, obuf0, obuf1, pos_v,
                sem_g0, sem_g1, sem_o0, sem_o1):
    rows = (rows0, rows1)
    obuf = (obuf0, obuf1)
    sem_g = (sem_g0, sem_g1)
    sem_o = (sem_o0, sem_o1)
    wid = lax.axis_index("s") * NC + lax.axis_index("c")
    b0 = wid * BBLK
    pltpu.sync_copy(pos_hbm, pos_v)
    pltpu.sync_copy(idx_hbm.at[:, pl.ds(b0, BBLK)], idx_v)
    lane = lax.iota(jnp.int32, LANES)

    def fire_gather(s, ph):
        pltpu.async_copy(table_hbm.at[idx_v.at[s]], rows[ph], sem_g[ph])

    def wait_gather(s, ph):
        pltpu.make_async_copy(table_hbm.at[idx_v.at[s]], rows[ph], sem_g[ph]).wait()

    def fire_out(s, ph):
        pltpu.async_copy(obuf[ph], out_hbm.at[s, :, pl.ds(b0, BBLK)], sem_o[ph])

    def wait_out(s, ph):
        pltpu.make_async_copy(
            obuf[ph], out_hbm.at[s, :, pl.ds(b0, BBLK)], sem_o[ph]
        ).wait()

    def compute(s, ph):
        rv = rows[ph]
        ov = obuf[ph]

        def dg_body(dg, c):
            pg = pos_v[s, pl.ds(dg * LANES, LANES)]
            for dl in range(LANES):
                d = dg * LANES + dl
                pv = jnp.broadcast_to(pg[dl], (LANES,))
                col = jnp.broadcast_to(d, (LANES,))
                for j in range(BG):
                    vals = plsc.load_gather(rv, [j * LANES + lane, col])
                    ov[d, pl.ds(j * LANES, LANES)] = vals + pv
            return c

        lax.fori_loop(0, DG, dg_body, 0)

    fire_gather(0, 0)

    def s2_body(s2, carry):
        for ph in range(2):
            s = 2 * s2 + ph

            @pl.when(s + 1 < S)
            def _():
                fire_gather(s + 1, 1 - ph)

            wait_gather(s, ph)

            @pl.when(s >= 2)
            def _():
                wait_out(s - 2, ph)

            compute(s, ph)
            fire_out(s, ph)
        return carry

    lax.fori_loop(0, S // 2, s2_body, 0)
    wait_out(S - 2, 0)
    wait_out(S - 1, 1)


def kernel(batch_seqs, item_emb, pos_weight):
    idx_t = batch_seqs.T
    table_p = jnp.pad(item_emb, ((0, 0), (0, DP - D)))
    k = pl.kernel(
        _embed_body,
        out_type=jax.ShapeDtypeStruct((S, D, B), jnp.float32),
        mesh=plsc.VectorSubcoreMesh(core_axis_name="c", subcore_axis_name="s"),
        compiler_params=pltpu.CompilerParams(
            use_tc_tiling_on_sc=True, needs_layout_passes=False
        ),
        scratch_types=[
            pltpu.VMEM((S, BBLK), jnp.int32),
            pltpu.VMEM((BBLK, DP), jnp.float32),
            pltpu.VMEM((BBLK, DP), jnp.float32),
            pltpu.VMEM((D, BBLK), jnp.float32),
            pltpu.VMEM((D, BBLK), jnp.float32),
            pltpu.VMEM((S, D), jnp.float32),
            pltpu.SemaphoreType.DMA,
            pltpu.SemaphoreType.DMA,
            pltpu.SemaphoreType.DMA,
            pltpu.SemaphoreType.DMA,
        ],
    )
    out_phys = k(idx_t, table_p, pos_weight)
    return jnp.transpose(out_phys, (2, 0, 1))


# speedup vs baseline: 3.3529x; 2.5546x over previous
import jax
import jax.numpy as jnp
from jax import lax
from jax.experimental import pallas as pl
from jax.experimental.pallas import tpu as pltpu
from jax.experimental.pallas import tpu_sc as plsc

B = 4096
S = 200
D = 64
DP = 128
NC = 2
NS = 16
NW = NC * NS
BBLK = B // NW
LANES = 16
DG = D // LANES
RUNROLL = 8


def _embed_body(idx_hbm, table_hbm, pos_hbm, out_hbm,
                idx_v, rows0, rows1, pos_v, sem_g0, sem_g1, sem_o0, sem_o1):
    rows = (rows0, rows1)
    sem_g = (sem_g0, sem_g1)
    sem_o = (sem_o0, sem_o1)
    wid = lax.axis_index("s") * NC + lax.axis_index("c")
    b0 = wid * BBLK
    pltpu.sync_copy(pos_hbm, pos_v)
    pltpu.sync_copy(idx_hbm.at[:, pl.ds(b0, BBLK)], idx_v)

    def fire_gather(s, ph):
        pltpu.async_copy(table_hbm.at[idx_v.at[s]], rows[ph], sem_g[ph])

    def wait_gather(s, ph):
        pltpu.make_async_copy(table_hbm.at[idx_v.at[s]], rows[ph], sem_g[ph]).wait()

    def fire_out(s, ph):
        pltpu.async_copy(rows[ph], out_hbm.at[pl.ds(b0, BBLK), s], sem_o[ph])

    def wait_out(s, ph):
        pltpu.make_async_copy(
            rows[ph], out_hbm.at[pl.ds(b0, BBLK), s], sem_o[ph]
        ).wait()

    def compute(s, ph):
        rv = rows[ph]
        pg = [pos_v[s, pl.ds(dg * LANES, LANES)] for dg in range(DG)]

        def r_body(rb, c):
            for ru in range(RUNROLL):
                r = rb * RUNROLL + ru
                for dg in range(DG):
                    plsc.addupdate(rv.at[r, pl.ds(dg * LANES, LANES)], pg[dg])
            return c

        lax.fori_loop(0, BBLK // RUNROLL, r_body, 0)

    fire_gather(0, 0)

    def s2_body(s2, carry):
        for ph in range(2):
            s = 2 * s2 + ph

            @pl.when(jnp.logical_and(s + 1 < S, s >= 1))
            def _():
                wait_out(s - 1, 1 - ph)

            @pl.when(s + 1 < S)
            def _():
                fire_gather(s + 1, 1 - ph)

            wait_gather(s, ph)
            compute(s, ph)
            fire_out(s, ph)
        return carry

    lax.fori_loop(0, S // 2, s2_body, 0)
    wait_out(S - 2, 0)
    wait_out(S - 1, 1)


def kernel(batch_seqs, item_emb, pos_weight):
    idx_t = batch_seqs.T
    table_p = jnp.pad(item_emb, ((0, 0), (0, DP - D)))
    k = pl.kernel(
        _embed_body,
        out_type=jax.ShapeDtypeStruct((B, S, DP), jnp.float32),
        mesh=plsc.VectorSubcoreMesh(core_axis_name="c", subcore_axis_name="s"),
        compiler_params=pltpu.CompilerParams(
            use_tc_tiling_on_sc=True, needs_layout_passes=False
        ),
        scratch_types=[
            pltpu.VMEM((S, BBLK), jnp.int32),
            pltpu.VMEM((BBLK, DP), jnp.float32),
            pltpu.VMEM((BBLK, DP), jnp.float32),
            pltpu.VMEM((S, D), jnp.float32),
            pltpu.SemaphoreType.DMA,
            pltpu.SemaphoreType.DMA,
            pltpu.SemaphoreType.DMA,
            pltpu.SemaphoreType.DMA,
        ],
    )
    out_p = k(idx_t, table_p, pos_weight)
    return out_p[:, :, :D]
